# baseline (speedup 1.0000x reference)
import functools

import jax
import jax.numpy as jnp
from jax.experimental import pallas as pl
from jax.experimental.pallas import tpu as pltpu

B, S, D, P, TOPK = 1024, 50, 128, 1024, 20
NEG_BIG = -3.0e38
MASK_FILL = -1.0e7

_SORT8 = [
    (0, 1), (2, 3), (4, 5), (6, 7),
    (0, 2), (1, 3), (4, 6), (5, 7),
    (1, 2), (5, 6),
    (0, 4), (1, 5), (2, 6), (3, 7),
    (2, 4), (3, 5),
    (1, 2), (3, 4), (5, 6),
]


def _sorted_cols(sim):
    lvl = [sim[:, 128 * j:128 * (j + 1)] for j in range(8)]
    for i, j in _SORT8:
        hi = jnp.maximum(lvl[i], lvl[j])
        lo = jnp.minimum(lvl[i], lvl[j])
        lvl[i], lvl[j] = hi, lo
    return lvl


def _pop_extract(lvl):
    nl = len(lvl)
    thresh = None
    for it in range(TOPK):
        thresh = jnp.max(lvl[0], axis=-1, keepdims=True)
        if it < TOPK - 1:
            popm = lvl[0] == thresh
            for j in range(nl - 1):
                lvl[j] = jnp.where(popm, lvl[j + 1], lvl[j])
            lvl[nl - 1] = jnp.where(popm, NEG_BIG, lvl[nl - 1])
    return thresh, lvl[0]


def _pop_extract_pairs(lvl):
    nl = len(lvl)
    sent = jnp.full_like(lvl[0], NEG_BIG)
    ext = lvl + [sent, sent]
    g2 = None
    h2 = None
    for it in range(TOPK // 2):
        h = ext[0]
        g1 = jnp.max(h, axis=-1, keepdims=True)
        pm1 = h == g1
        h2 = jnp.where(pm1, ext[1], h)
        g2 = jnp.max(h2, axis=-1, keepdims=True)
        if it < TOPK // 2 - 1:
            pm2 = h2 == g2
            s1 = pm1 ^ pm2
            s2 = pm1 & pm2
            for j in range(nl):
                ext[j] = jnp.where(
                    s1, ext[j + 1], jnp.where(s2, ext[j + 2], ext[j]))
    return g2, h2


def _fused_kernel(ss_ref, proto_ref, bool_ref, emb_ref, t_ref, *, bb):
    rows = bb * S
    ss = ss_ref[...].reshape(rows, D)
    proto = proto_ref[...]
    sim = jax.lax.dot_general(
        ss, proto,
        dimension_numbers=(((1,), (1,)), ((), ())),
        preferred_element_type=jnp.float32,
    )

    lvl = _sorted_cols(sim)
    thresh, heads = _pop_extract_pairs(lvl[:5])
    t_ref[...] = thresh
    need_full = jnp.min(heads) < jnp.float32(-1.0e37)

    @pl.when(need_full)
    def _fallback():
        full_t, _ = _pop_extract(_sorted_cols(sim))
        t_ref[...] = full_t

    sim3 = sim.reshape(bb, S, P)
    t3 = t_ref[...].reshape(bb, S, 1)
    in_topk = (sim3 >= t3).astype(jnp.float32)
    cnt = jnp.sum(in_topk, axis=1)
    mask = cnt >= jnp.float32(S)
    mean = jnp.mean(sim3, axis=1)
    masked = jnp.where(mask, mean, jnp.float32(MASK_FILL))
    m = jnp.max(masked, axis=-1, keepdims=True)
    e = jnp.exp(masked - m)
    dist = e / jnp.sum(e, axis=-1, keepdims=True)
    emb = jax.lax.dot_general(
        dist, proto,
        dimension_numbers=(((1,), (0,)), ((), ())),
        preferred_element_type=jnp.float32,
    )
    norm = jnp.sqrt(jnp.sum(emb * emb, axis=-1, keepdims=True))
    emb = emb / jnp.maximum(norm, jnp.float32(1e-12))
    bool_ref[...] = mask
    emb_ref[...] = emb


def kernel(support_sets, proto_embs):
    bb = 16
    grid = (B // bb,)
    f = functools.partial(_fused_kernel, bb=bb)
    out_bool, out_emb = pl.pallas_call(
        f,
        grid=grid,
        in_specs=[
            pl.BlockSpec((bb, S, D), lambda i: (i, 0, 0)),
            pl.BlockSpec((P, D), lambda i: (0, 0)),
        ],
        out_specs=[
            pl.BlockSpec((bb, P), lambda i: (i, 0)),
            pl.BlockSpec((bb, D), lambda i: (i, 0)),
        ],
        out_shape=[
            jax.ShapeDtypeStruct((B, P), jnp.bool_),
            jax.ShapeDtypeStruct((B, D), jnp.float32),
        ],
        scratch_shapes=[pltpu.VMEM((bb * S, 1), jnp.float32)],
    )(support_sets, proto_embs)
    return out_bool, out_emb


# speedup vs baseline: 1.0374x; 1.0374x over previous
import functools

import jax
import jax.numpy as jnp
from jax.experimental import pallas as pl
from jax.experimental.pallas import tpu as pltpu

B, S, D, P, TOPK = 1024, 50, 128, 1024, 20
NEG_BIG = -3.0e38
MASK_FILL = -1.0e7

_SORT8 = [
    (0, 1), (2, 3), (4, 5), (6, 7),
    (0, 2), (1, 3), (4, 6), (5, 7),
    (1, 2), (5, 6),
    (0, 4), (1, 5), (2, 6), (3, 7),
    (2, 4), (3, 5),
    (1, 2), (3, 4), (5, 6),
]


def _sorted_cols(sim):
    lvl = [sim[:, 128 * j:128 * (j + 1)] for j in range(8)]
    for i, j in _SORT8:
        hi = jnp.maximum(lvl[i], lvl[j])
        lo = jnp.minimum(lvl[i], lvl[j])
        lvl[i], lvl[j] = hi, lo
    return lvl


def _pop_extract(lvl):
    nl = len(lvl)
    thresh = None
    for it in range(TOPK):
        thresh = jnp.max(lvl[0], axis=-1, keepdims=True)
        if it < TOPK - 1:
            popm = lvl[0] == thresh
            for j in range(nl - 1):
                lvl[j] = jnp.where(popm, lvl[j + 1], lvl[j])
            lvl[nl - 1] = jnp.where(popm, NEG_BIG, lvl[nl - 1])
    return thresh, lvl[0]


def _fused_kernel(ss_ref, proto_ref, bool_ref, emb_ref, t_ref, *, bb):
    rows = bb * S
    ss = ss_ref[...].reshape(rows, D)
    proto = proto_ref[...]
    sim = jax.lax.dot_general(
        ss, proto,
        dimension_numbers=(((1,), (1,)), ((), ())),
        preferred_element_type=jnp.float32,
    )

    lvl = _sorted_cols(sim)
    thresh, heads = _pop_extract(lvl[:5])
    t_ref[...] = thresh
    need_full = jnp.min(heads) < jnp.float32(-1.0e37)

    @pl.when(need_full)
    def _fallback():
        full_t, _ = _pop_extract(_sorted_cols(sim))
        t_ref[...] = full_t

    sim3 = sim.reshape(bb, S, P)
    t3 = t_ref[...].reshape(bb, S, 1)
    in_topk = (sim3 >= t3).astype(jnp.float32)
    cnt = jnp.sum(in_topk, axis=1)
    mask = cnt >= jnp.float32(S)
    mean = jnp.mean(sim3, axis=1)
    masked = jnp.where(mask, mean, jnp.float32(MASK_FILL))
    m = jnp.max(masked, axis=-1, keepdims=True)
    e = jnp.exp(masked - m)
    dist = e / jnp.sum(e, axis=-1, keepdims=True)
    emb = jax.lax.dot_general(
        dist, proto,
        dimension_numbers=(((1,), (0,)), ((), ())),
        preferred_element_type=jnp.float32,
    )
    norm = jnp.sqrt(jnp.sum(emb * emb, axis=-1, keepdims=True))
    emb = emb / jnp.maximum(norm, jnp.float32(1e-12))
    bool_ref[...] = mask
    emb_ref[...] = emb


def kernel(support_sets, proto_embs):
    bb = 16
    grid = (B // bb,)
    f = functools.partial(_fused_kernel, bb=bb)
    out_bool, out_emb = pl.pallas_call(
        f,
        grid=grid,
        in_specs=[
            pl.BlockSpec((bb, S, D), lambda i: (i, 0, 0)),
            pl.BlockSpec((P, D), lambda i: (0, 0)),
        ],
        out_specs=[
            pl.BlockSpec((bb, P), lambda i: (i, 0)),
            pl.BlockSpec((bb, D), lambda i: (i, 0)),
        ],
        out_shape=[
            jax.ShapeDtypeStruct((B, P), jnp.bool_),
            jax.ShapeDtypeStruct((B, D), jnp.float32),
        ],
        scratch_shapes=[pltpu.VMEM((bb * S, 1), jnp.float32)],
    )(support_sets, proto_embs)
    return out_bool, out_emb


# speedup vs baseline: 1.1006x; 1.0610x over previous
import functools

import jax
import jax.numpy as jnp
from jax.experimental import pallas as pl
from jax.experimental.pallas import tpu as pltpu

B, S, D, P, TOPK = 1024, 50, 128, 1024, 20
NEG_BIG = -3.0e38
MASK_FILL = -1.0e7

_SORT8 = [
    (0, 1), (2, 3), (4, 5), (6, 7),
    (0, 2), (1, 3), (4, 6), (5, 7),
    (1, 2), (5, 6),
    (0, 4), (1, 5), (2, 6), (3, 7),
    (2, 4), (3, 5),
    (1, 2), (3, 4), (5, 6),
]


def _sorted_cols(sim):
    lvl = [sim[:, 128 * j:128 * (j + 1)] for j in range(8)]
    for i, j in _SORT8:
        hi = jnp.maximum(lvl[i], lvl[j])
        lo = jnp.minimum(lvl[i], lvl[j])
        lvl[i], lvl[j] = hi, lo
    return lvl


def _pop_extract(lvl):
    nl = len(lvl)
    thresh = None
    for it in range(TOPK):
        thresh = jnp.max(lvl[0], axis=-1, keepdims=True)
        if it < TOPK - 1:
            popm = lvl[0] == thresh
            for j in range(nl - 1):
                lvl[j] = jnp.where(popm, lvl[j + 1], lvl[j])
            lvl[nl - 1] = jnp.where(popm, NEG_BIG, lvl[nl - 1])
    return thresh, lvl[0]


def _fused_kernel(ss_ref, proto_ref, bool_ref, emb_ref, t_ref, *, bb):
    rows = bb * S
    ss = ss_ref[...].reshape(rows, D)
    proto = proto_ref[...]
    sim = jax.lax.dot_general(
        ss, proto,
        dimension_numbers=(((1,), (1,)), ((), ())),
        preferred_element_type=jnp.float32,
    )

    lvl = _sorted_cols(sim)
    thresh, heads = _pop_extract(lvl[:5])
    t_ref[...] = thresh
    need_full = jnp.min(heads) < jnp.float32(-1.0e37)

    @pl.when(need_full)
    def _fallback():
        full_t, _ = _pop_extract(_sorted_cols(sim))
        t_ref[...] = full_t

    sim3 = sim.reshape(bb, S, P)
    t3 = t_ref[...].reshape(bb, S, 1)
    in_topk = (sim3 >= t3).astype(jnp.float32)
    cnt = jnp.sum(in_topk, axis=1)
    mask = cnt >= jnp.float32(S)
    ssm = jnp.mean(ss_ref[...], axis=1)
    mean = jax.lax.dot_general(
        ssm, proto,
        dimension_numbers=(((1,), (1,)), ((), ())),
        preferred_element_type=jnp.float32,
    )
    masked = jnp.where(mask, mean, jnp.float32(MASK_FILL))
    m = jnp.max(masked, axis=-1, keepdims=True)
    e = jnp.exp(masked - m)
    dist = e / jnp.sum(e, axis=-1, keepdims=True)
    emb = jax.lax.dot_general(
        dist, proto,
        dimension_numbers=(((1,), (0,)), ((), ())),
        preferred_element_type=jnp.float32,
    )
    norm = jnp.sqrt(jnp.sum(emb * emb, axis=-1, keepdims=True))
    emb = emb / jnp.maximum(norm, jnp.float32(1e-12))
    bool_ref[...] = mask
    emb_ref[...] = emb


def kernel(support_sets, proto_embs):
    bb = 16
    grid = (B // bb,)
    f = functools.partial(_fused_kernel, bb=bb)
    out_bool, out_emb = pl.pallas_call(
        f,
        grid=grid,
        in_specs=[
            pl.BlockSpec((bb, S, D), lambda i: (i, 0, 0)),
            pl.BlockSpec((P, D), lambda i: (0, 0)),
        ],
        out_specs=[
            pl.BlockSpec((bb, P), lambda i: (i, 0)),
            pl.BlockSpec((bb, D), lambda i: (i, 0)),
        ],
        out_shape=[
            jax.ShapeDtypeStruct((B, P), jnp.bool_),
            jax.ShapeDtypeStruct((B, D), jnp.float32),
        ],
        scratch_shapes=[pltpu.VMEM((bb * S, 1), jnp.float32)],
    )(support_sets, proto_embs)
    return out_bool, out_emb
